# baseline (speedup 1.0000x reference)
import functools

import jax
import jax.numpy as jnp
from jax import lax
from jax.experimental import pallas as pl
from jax.experimental.pallas import tpu as pltpu
from jax.experimental.pallas import tpu_sc as plsc

_B = 16384
_D = 64
_NC = 2
_NS = 16
_NW = _NC * _NS
_RPT = _B // _NW
_CH = 128
_NCHUNK = _RPT // _CH

_mesh = plsc.VectorSubcoreMesh(core_axis_name="c", subcore_axis_name="s")


@functools.partial(
    pl.kernel,
    mesh=_mesh,
    compiler_params=pltpu.CompilerParams(
        needs_layout_passes=False,
        use_tc_tiling_on_sc=True,
        disable_bounds_checks=True,
    ),
    out_type=jax.ShapeDtypeStruct((_B,), jnp.float32),
    scratch_types=[
        pltpu.VMEM((_RPT,), jnp.int32),
        pltpu.VMEM((_RPT,), jnp.int32),
        pltpu.VMEM((_RPT,), jnp.int32),
        pltpu.VMEM((2, _CH), jnp.int32),
        pltpu.VMEM((2, _CH), jnp.int32),
        pltpu.VMEM((2, _CH), jnp.int32),
        pltpu.VMEM((2, _CH, 128), jnp.float32),
        pltpu.VMEM((2, _CH, 128), jnp.float32),
        pltpu.VMEM((2, _CH, 128), jnp.float32),
        pltpu.VMEM((_RPT,), jnp.float32),
        pltpu.SemaphoreType.DMA((2,)),
    ],
)
def _sc_scores(uid_hbm, pid_hbm, nid_hbm, utab_hbm, itab_hbm, out_hbm,
               idx_u, idx_p, idx_n, six_u, six_p, six_n,
               dat_u, dat_p, dat_n, scores, sem):
    wid = lax.axis_index("s") * _NC + lax.axis_index("c")
    base = wid * _RPT

    pltpu.sync_copy(uid_hbm.at[pl.ds(base, _RPT)], idx_u)
    pltpu.sync_copy(pid_hbm.at[pl.ds(base, _RPT)], idx_p)
    pltpu.sync_copy(nid_hbm.at[pl.ds(base, _RPT)], idx_n)

    def fire(c, par):
        for ids, six, tab, dat in (
            (idx_u, six_u, utab_hbm, dat_u),
            (idx_p, six_p, itab_hbm, dat_p),
            (idx_n, six_n, itab_hbm, dat_n),
        ):
            for k in range(_CH // 16):
                v = ids[pl.ds(c * _CH + k * 16, 16)]
                six[par, pl.ds(k * 16, 16)] = v >> 1
            pltpu.async_copy(tab.at[six.at[par]], dat.at[par], sem.at[par])

    def drain(par):
        s = sem.at[par]
        pltpu.make_async_copy(utab_hbm.at[six_u.at[par]], dat_u.at[par], s).wait()
        pltpu.make_async_copy(itab_hbm.at[six_p.at[par]], dat_p.at[par], s).wait()
        pltpu.make_async_copy(itab_hbm.at[six_n.at[par]], dat_n.at[par], s).wait()

    def compute(c, par):
        lane = lax.iota(jnp.int32, 16)
        for k in range(_CH // 16):
            uoff = (idx_u[pl.ds(c * _CH + k * 16, 16)] & 1) * 64
            poff = (idx_p[pl.ds(c * _CH + k * 16, 16)] & 1) * 64
            noff = (idx_n[pl.ds(c * _CH + k * 16, 16)] & 1) * 64
            tvec = jnp.zeros((16,), jnp.float32)
            for l in range(16):
                slot = k * 16 + l
                ub, pb, nb = uoff[l], poff[l], noff[l]
                acc = jnp.zeros((16,), jnp.float32)
                for q in range(_D // 16):
                    u = dat_u[par, slot, pl.ds(ub + q * 16, 16)]
                    p = dat_p[par, slot, pl.ds(pb + q * 16, 16)]
                    n = dat_n[par, slot, pl.ds(nb + q * 16, 16)]
                    acc = acc + u * (p - n)
                tvec = jnp.where(lane == l, jnp.sum(acc), tvec)
            scores[pl.ds(c * _CH + k * 16, 16)] = tvec

    fire(0, 0)

    def body(c, carry):
        par = c & 1
        fire(c, par)
        drain(1 - par)
        compute(c - 1, 1 - par)
        return carry

    lax.fori_loop(1, _NCHUNK, body, 0, unroll=False)
    drain(_NCHUNK & 1 ^ 1)
    compute(_NCHUNK - 1, _NCHUNK & 1 ^ 1)

    pltpu.sync_copy(scores, out_hbm.at[pl.ds(base, _RPT)])


def _loss_body(x_ref, o_ref):
    x = x_ref[...]
    z = jnp.exp(-jnp.abs(x))
    ls = jnp.minimum(x, 0.0) - jnp.log(1.0 + z)
    o_ref[0, 0] = -jnp.sum(ls)


def kernel(user_id, pos_id, neg_id, user_table, item_table):
    ut2 = user_table.reshape(500000, 128)
    it2 = item_table.reshape(500000, 128)
    tmp = _sc_scores(user_id, pos_id, neg_id, ut2, it2)
    loss = pl.pallas_call(
        _loss_body,
        out_shape=jax.ShapeDtypeStruct((1, 1), jnp.float32),
        out_specs=pl.BlockSpec(memory_space=pltpu.SMEM),
    )(tmp.reshape(128, 128))
    return loss[0, 0]
